# Initial kernel scaffold; baseline (speedup 1.0000x reference)
import jax
import jax.numpy as jnp
from jax.experimental import pallas as pl


def kernel(d_vals, sdf):
    raise NotImplementedError("write your pallas kernel here")


# speedup vs baseline: 4.0339x; 4.0339x over previous
import functools

import jax
import jax.numpy as jnp
from jax import lax
from jax.experimental import pallas as pl
from jax.experimental.pallas import tpu as pltpu
from jax.experimental.pallas import tpu_sc as plsc

ALPHA = 10.0
BETA = 0.1
N_PTS = 128
N_IMP = 64
L = 16
CH = 128


def _make_sc_kernel(n_rays):
    info = plsc.get_sparse_core_info()
    nw = info.num_cores * info.num_subcores
    rays_per_w = n_rays // nw
    n_chunks = rays_per_w // CH
    mesh = plsc.VectorSubcoreMesh(core_axis_name="c", subcore_axis_name="s")

    @functools.partial(
        pl.kernel,
        mesh=mesh,
        out_type=jax.ShapeDtypeStruct((n_rays * N_IMP,), jnp.float32),
        scratch_types=[
            pltpu.VMEM((CH * N_PTS,), jnp.float32),
            pltpu.VMEM((CH * N_PTS,), jnp.float32),
            pltpu.VMEM((CH * N_IMP,), jnp.float32),
            pltpu.VMEM((N_PTS,), jnp.float32),
        ],
        compiler_params=pltpu.CompilerParams(needs_layout_passes=False),
    )
    def k(d_hbm, sdf_hbm, out_hbm, dbuf, sbuf, obuf, cdfbuf):
        wid = lax.axis_index("s") * info.num_cores + lax.axis_index("c")
        iota = lax.iota(jnp.int32, L)

        def ray_body(r, _):
            rbase = r * N_PTS
            carry = jnp.float32(0.0)
            for kk in range(N_PTS // L):
                dk = dbuf[pl.ds(rbase + L * kk, L)]
                sdfk = sbuf[pl.ds(rbase + L * kk, L)]
                e = jnp.exp(jnp.abs(sdfk) * jnp.float32(-1.0 / BETA))
                a = jnp.float32(0.5 * ALPHA) * e
                sigma = jnp.where(sdfk >= 0, a, jnp.float32(ALPHA) - a)
                sh_idx = rbase + jnp.minimum(iota + (L * kk + 1), N_PTS - 1)
                dk1 = plsc.load_gather(dbuf, [sh_idx])
                s = sigma * (dk1 - dk)
                inc = plsc.cumsum(s)
                cdfk = jnp.float32(1.0) - jnp.exp(-(inc - s + carry))
                cdfbuf[pl.ds(L * kk, L)] = cdfk
                carry = carry + jnp.sum(s, axis=0)

            for qc in range(N_IMP // L):
                uq = (iota + (L * qc)).astype(jnp.float32) * jnp.float32(1.0 / 63.0)
                inds = jnp.zeros((L,), jnp.int32)
                for w in (64, 32, 16, 8, 4, 2, 1):
                    g = plsc.load_gather(cdfbuf, [inds + (w - 1)])
                    inds = jnp.where(g < uq, inds + w, inds)
                below = jnp.maximum(inds - 1, 0)
                above = jnp.minimum(inds, N_PTS - 2)
                g0 = plsc.load_gather(cdfbuf, [below])
                g1 = plsc.load_gather(cdfbuf, [above])
                b0 = plsc.load_gather(dbuf, [rbase + below])
                b1 = plsc.load_gather(dbuf, [rbase + above])
                den = g1 - g0
                den = jnp.where(den < jnp.float32(1e-5), jnp.float32(1.0), den)
                tq = (uq - g0) / den
                obuf[pl.ds(r * N_IMP + L * qc, L)] = b0 + tq * (b1 - b0)
            return 0

        def chunk_body(g, _):
            row0 = wid * rays_per_w + g * CH
            pltpu.sync_copy(d_hbm.at[pl.ds(row0 * N_PTS, CH * N_PTS)], dbuf)
            pltpu.sync_copy(sdf_hbm.at[pl.ds(row0 * N_PTS, CH * N_PTS)], sbuf)
            lax.fori_loop(0, CH, ray_body, 0)
            pltpu.sync_copy(obuf, out_hbm.at[pl.ds(row0 * N_IMP, CH * N_IMP)])
            return 0

        lax.fori_loop(0, n_chunks, chunk_body, 0)

    return k


@jax.jit
def kernel(d_vals, sdf):
    n_rays = d_vals.shape[0]
    k = _make_sc_kernel(n_rays)
    out = k(d_vals.reshape(-1), sdf.reshape(-1))
    return out.reshape(n_rays, N_IMP)
